# Initial kernel scaffold; baseline (speedup 1.0000x reference)
import jax
import jax.numpy as jnp
from jax.experimental import pallas as pl


def kernel(x_shot, x_query, W_enc, b_enc, r_cos, r_dn4):
    raise NotImplementedError("write your pallas kernel here")


# speedup vs baseline: 17.4184x; 17.4184x over previous
import jax
import jax.numpy as jnp
from jax.experimental import pallas as pl
from jax.experimental.pallas import tpu as pltpu

_K = 5
_DIM = 384
_HW = 196
_PATCH = 768
_WAY = 5
_SHOT = 5
_NQ = 75
_NQ_PAD = 80
_QB = 8
_ROWS = _QB * _HW
_SHW = _SHOT * _HW
_IMG_BLK = 4
_ENC_ROWS = _IMG_BLK * _HW


def _encode_body(p_ref, w_ref, b_ref, f_ref):
    f = jax.lax.dot_general(
        p_ref[...], w_ref[...], (((1,), (0,)), ((), ())),
        preferred_element_type=jnp.float32)
    f_ref[...] = jnp.maximum(f + b_ref[...], 0.0)


def _dn4_body(qf_ref, bt_ref, rcos_ref, rdn4_ref, out_ref):
    bt = bt_ref[0]
    colsq = jnp.sum(bt * bt, axis=1, keepdims=True)
    snt = bt * jax.lax.rsqrt(colsq)
    proto = jnp.sum(bt, axis=1, keepdims=True) * (1.0 / _SHW)
    pn = proto * jax.lax.rsqrt(
        jnp.maximum(jnp.sum(proto * proto), 1e-24))

    qf = qf_ref[...]
    inner = jax.lax.dot_general(
        qf, snt, (((1,), (0,)), ((), ())),
        preferred_element_type=jnp.float32)

    x = inner
    acc = jnp.zeros((_ROWS, 1), jnp.float32)
    k = jnp.full((_ROWS, 1), float(_K), jnp.float32)
    for _ in range(_K):
        m = jnp.max(x, axis=1, keepdims=True)
        eq = x == m
        c = jnp.sum(eq.astype(jnp.float32), axis=1, keepdims=True)
        t = jnp.minimum(c, k)
        acc = acc + m * t
        k = k - t
        x = jnp.where(eq, -3.4e38, x)

    rowsq = jnp.sum(qf * qf, axis=1, keepdims=True)
    rowinv = jnp.where(rowsq > 0.0, jax.lax.rsqrt(rowsq), 0.0)
    acc = acc * rowinv

    rid = jax.lax.broadcasted_iota(jnp.int32, (_QB, _ROWS), 1) // _HW
    qid = jax.lax.broadcasted_iota(jnp.int32, (_QB, _ROWS), 0)
    sel = (rid == qid).astype(jnp.float32)
    dn4 = jax.lax.dot_general(
        sel, acc, (((1,), (0,)), ((), ())),
        preferred_element_type=jnp.float32)

    qm = jax.lax.dot_general(
        sel, qf, (((1,), (0,)), ((), ())),
        preferred_element_type=jnp.float32) * (1.0 / _HW)
    qmsq = jnp.sum(qm * qm, axis=1, keepdims=True)
    qmn = qm * jax.lax.rsqrt(jnp.maximum(qmsq, 1e-24))
    cos = jax.lax.dot_general(
        qmn, pn, (((1,), (0,)), ((), ())),
        preferred_element_type=jnp.float32)

    val = rcos_ref[0] * cos + (rdn4_ref[0] * (1.0 / _K)) * dn4
    out_ref[0] = jnp.broadcast_to(val, (_QB, 128))


def kernel(x_shot, x_query, W_enc, b_enc, r_cos, r_dn4):
    ep = x_shot.shape[0]
    xs = x_shot.reshape((-1, 3, 224, 224))
    xq = x_query.reshape((-1, 3, 224, 224))
    x = jnp.concatenate([xs, xq], axis=0)
    n = x.shape[0]
    patches = x.reshape(n, 3, 14, 16, 14, 16).transpose(0, 2, 4, 1, 3, 5)
    patches = patches.reshape(n * _HW, _PATCH)
    wt = W_enc.reshape(_DIM, _PATCH).T
    b2 = b_enc.reshape(1, _DIM)

    feats = pl.pallas_call(
        _encode_body,
        grid=(n * _HW // _ENC_ROWS,),
        in_specs=[
            pl.BlockSpec((_ENC_ROWS, _PATCH), lambda i: (i, 0)),
            pl.BlockSpec((_PATCH, _DIM), lambda i: (0, 0)),
            pl.BlockSpec((1, _DIM), lambda i: (0, 0)),
        ],
        out_specs=pl.BlockSpec((_ENC_ROWS, _DIM), lambda i: (i, 0)),
        out_shape=jax.ShapeDtypeStruct((n * _HW, _DIM), jnp.float32),
    )(patches, wt, b2)

    n_s = _WAY * _SHOT * _HW
    bmix = feats[:n_s].reshape(_WAY, _SHW, _DIM)
    bt = jnp.swapaxes(bmix, 1, 2)
    qf = feats[n_s:]
    qf = jnp.pad(qf, ((0, (_NQ_PAD - _NQ) * _HW), (0, 0)))

    out = pl.pallas_call(
        _dn4_body,
        grid=(_WAY, _NQ_PAD // _QB),
        in_specs=[
            pl.BlockSpec((_ROWS, _DIM), lambda w, q: (q, 0)),
            pl.BlockSpec((1, _DIM, _SHW), lambda w, q: (w, 0, 0)),
            pl.BlockSpec(memory_space=pltpu.SMEM),
            pl.BlockSpec(memory_space=pltpu.SMEM),
        ],
        out_specs=pl.BlockSpec((1, _QB, 128), lambda w, q: (w, q, 0)),
        out_shape=jax.ShapeDtypeStruct((_WAY, _NQ_PAD, 128), jnp.float32),
    )(qf, bt, r_cos, r_dn4)

    logits = out[:, :_NQ, 0].T
    return logits.reshape(ep, _NQ, _WAY)


# speedup vs baseline: 19.5220x; 1.1208x over previous
import jax
import jax.numpy as jnp
from jax.experimental import pallas as pl
from jax.experimental.pallas import tpu as pltpu

_K = 5
_DIM = 384
_HW = 196
_PATCH = 768
_WAY = 5
_SHOT = 5
_NQ = 75
_NQ_PAD = 80
_QB = 8
_COLS = 1664
_SHW = _SHOT * _HW
_SHW_PAD = 984
_CHUNK = 8
_NCHUNK = _SHW_PAD // _CHUNK
_IMG_BLK = 4
_ENC_ROWS = _IMG_BLK * _HW


def _encode_body(p_ref, w_ref, b_ref, f_ref):
    f = jax.lax.dot_general(
        p_ref[...], w_ref[...], (((1,), (0,)), ((), ())),
        preferred_element_type=jnp.float32)
    f_ref[...] = jnp.maximum(f + b_ref[...], 0.0)


def _dn4_body(bmix_ref, qft_ref, rcos_ref, rdn4_ref, out_ref, sn_s, pn_s):
    q = pl.program_id(1)

    @pl.when(q == 0)
    def _init():
        b = bmix_ref[0]
        colsq = jnp.sum(b * b, axis=0, keepdims=True)
        sn = b * jax.lax.rsqrt(colsq)
        sn_s[...] = jnp.concatenate(
            [sn, jnp.zeros((_SHW_PAD - _SHW, _DIM), jnp.float32)], axis=0)
        proto = jnp.sum(b, axis=0, keepdims=True) * (1.0 / _SHW)
        pn_s[...] = proto * jax.lax.rsqrt(
            jnp.maximum(jnp.sum(proto * proto), 1e-24))

    qft = qft_ref[...]
    innerT = jax.lax.dot_general(
        sn_s[...], qft, (((1,), (0,)), ((), ())),
        preferred_element_type=jnp.float32)

    r = [jnp.zeros((_CHUNK, _COLS), jnp.float32) for _ in range(_K)]
    for c in range(_NCHUNK):
        v = jax.lax.slice(innerT, (c * _CHUNK, 0), ((c + 1) * _CHUNK, _COLS))
        for j in range(_K):
            nr = jnp.maximum(r[j], v)
            if j + 1 < _K:
                v = jnp.minimum(r[j], v)
            r[j] = nr
    cand = jnp.concatenate(r, axis=0)

    acc = jnp.zeros((1, _COLS), jnp.float32)
    k = jnp.full((1, _COLS), float(_K), jnp.float32)
    x = cand
    for _ in range(_K):
        m = jnp.max(x, axis=0, keepdims=True)
        eq = x == m
        cnt = jnp.sum(eq.astype(jnp.float32), axis=0, keepdims=True)
        t = jnp.minimum(cnt, k)
        acc = acc + m * t
        k = k - t
        x = jnp.where(eq, -3.4e38, x)

    rowsq = jnp.sum(qft * qft, axis=0, keepdims=True)
    accq = acc * jnp.where(rowsq > 0.0, jax.lax.rsqrt(rowsq), 0.0)

    rid = jax.lax.broadcasted_iota(jnp.int32, (_QB, _COLS), 1) // _HW
    qid = jax.lax.broadcasted_iota(jnp.int32, (_QB, _COLS), 0)
    sel = (rid == qid).astype(jnp.float32)
    dn4 = jax.lax.dot_general(
        sel, jnp.transpose(accq), (((1,), (0,)), ((), ())),
        preferred_element_type=jnp.float32)

    qm = jax.lax.dot_general(
        qft, jnp.transpose(sel), (((1,), (0,)), ((), ())),
        preferred_element_type=jnp.float32) * (1.0 / _HW)
    qmsq = jnp.sum(qm * qm, axis=0, keepdims=True)
    qmn = qm * jax.lax.rsqrt(jnp.maximum(qmsq, 1e-24))
    cos = jax.lax.dot_general(
        pn_s[...], qmn, (((1,), (0,)), ((), ())),
        preferred_element_type=jnp.float32)

    val = rcos_ref[0] * jnp.transpose(cos) + (rdn4_ref[0] * (1.0 / _K)) * dn4
    out_ref[0] = jnp.broadcast_to(val, (_QB, 128))


def kernel(x_shot, x_query, W_enc, b_enc, r_cos, r_dn4):
    ep = x_shot.shape[0]
    xs = x_shot.reshape((-1, 3, 224, 224))
    xq = x_query.reshape((-1, 3, 224, 224))
    x = jnp.concatenate([xs, xq], axis=0)
    n = x.shape[0]
    patches = x.reshape(n, 3, 14, 16, 14, 16).transpose(0, 2, 4, 1, 3, 5)
    patches = patches.reshape(n * _HW, _PATCH)
    wt = W_enc.reshape(_DIM, _PATCH).T
    b2 = b_enc.reshape(1, _DIM)

    feats = pl.pallas_call(
        _encode_body,
        grid=(n * _HW // _ENC_ROWS,),
        in_specs=[
            pl.BlockSpec((_ENC_ROWS, _PATCH), lambda i: (i, 0)),
            pl.BlockSpec((_PATCH, _DIM), lambda i: (0, 0)),
            pl.BlockSpec((1, _DIM), lambda i: (0, 0)),
        ],
        out_specs=pl.BlockSpec((_ENC_ROWS, _DIM), lambda i: (i, 0)),
        out_shape=jax.ShapeDtypeStruct((n * _HW, _DIM), jnp.float32),
    )(patches, wt, b2)

    n_s = _WAY * _SHOT * _HW
    bmix = feats[:n_s].reshape(_WAY, _SHW, _DIM)
    qf = feats[n_s:]
    qf = jnp.pad(qf, ((0, (_NQ_PAD - _NQ) * _HW), (0, 0)))
    qfb = qf.reshape(_NQ_PAD // _QB, _QB * _HW, _DIM)
    qfb = jnp.pad(qfb, ((0, 0), (0, _COLS - _QB * _HW), (0, 0)))
    qft = qfb.reshape(_NQ_PAD // _QB * _COLS, _DIM).T

    out = pl.pallas_call(
        _dn4_body,
        grid=(_WAY, _NQ_PAD // _QB),
        in_specs=[
            pl.BlockSpec((1, _SHW, _DIM), lambda w, q: (w, 0, 0)),
            pl.BlockSpec((_DIM, _COLS), lambda w, q: (0, q)),
            pl.BlockSpec(memory_space=pltpu.SMEM),
            pl.BlockSpec(memory_space=pltpu.SMEM),
        ],
        out_specs=pl.BlockSpec((1, _QB, 128), lambda w, q: (w, q, 0)),
        out_shape=jax.ShapeDtypeStruct((_WAY, _NQ_PAD, 128), jnp.float32),
        scratch_shapes=[
            pltpu.VMEM((_SHW_PAD, _DIM), jnp.float32),
            pltpu.VMEM((1, _DIM), jnp.float32),
        ],
    )(bmix, qft, r_cos, r_dn4)

    logits = out[:, :_NQ, 0].T
    return logits.reshape(ep, _NQ, _WAY)


# speedup vs baseline: 41.8798x; 2.1453x over previous
import jax
import jax.numpy as jnp
from jax.experimental import pallas as pl
from jax.experimental.pallas import tpu as pltpu

_K = 5
_DIM = 384
_HW = 196
_PATCH = 768
_WAY = 5
_SHOT = 5
_NQ = 75
_NQ_PAD = 80
_QB = 8
_QROWS = _QB * _HW
_COLS = 1664
_SHW = _SHOT * _HW
_SHW_PAD = 984
_CHUNK = 8
_NCHUNK = _SHW_PAD // _CHUNK


def _enc_support_body(p_ref, w_ref, b_ref, sn_ref, pn_ref):
    f = jax.lax.dot_general(
        p_ref[...], w_ref[...], (((1,), (0,)), ((), ())),
        preferred_element_type=jnp.float32)
    f = jnp.maximum(f + b_ref[...], 0.0)
    for w in range(_WAY):
        fw = jax.lax.slice(f, (_SHW * w, 0), (_SHW * (w + 1), _DIM))
        colsq = jnp.sum(fw * fw, axis=0, keepdims=True)
        snw = fw * jax.lax.rsqrt(colsq)
        snw = jnp.concatenate(
            [snw, jnp.zeros((_SHW_PAD - _SHW, _DIM), jnp.float32)], axis=0)
        sn_ref[w] = snw.astype(jnp.bfloat16)
        proto = jnp.sum(fw, axis=0, keepdims=True) * (1.0 / _SHW)
        pn = proto * jax.lax.rsqrt(
            jnp.maximum(jnp.sum(proto * proto), 1e-24))
        pn_ref[w] = jnp.broadcast_to(pn, (8, _DIM))


def _enc_query_body(p_ref, w_ref, b_ref, qt_ref, ri_ref, qm_ref):
    f = jax.lax.dot_general(
        p_ref[...], w_ref[...], (((1,), (0,)), ((), ())),
        preferred_element_type=jnp.float32)
    f = jnp.maximum(f + b_ref[...], 0.0)
    ft = jnp.transpose(f)
    qt_ref[...] = jnp.concatenate(
        [ft, jnp.zeros((_DIM, _COLS - _QROWS), jnp.float32)],
        axis=1).astype(jnp.bfloat16)
    rowsq = jnp.sum(ft * ft, axis=0, keepdims=True)
    ri = jnp.where(rowsq > 0.0, jax.lax.rsqrt(rowsq), 0.0)
    ri = jnp.concatenate(
        [ri, jnp.zeros((1, _COLS - _QROWS), jnp.float32)], axis=1)
    ri_ref[0] = jnp.broadcast_to(ri, (8, _COLS))
    rid = jax.lax.broadcasted_iota(jnp.int32, (_QB, _QROWS), 1) // _HW
    qid = jax.lax.broadcasted_iota(jnp.int32, (_QB, _QROWS), 0)
    sel = (rid == qid).astype(jnp.float32)
    qm = jax.lax.dot_general(
        sel, f, (((1,), (0,)), ((), ())),
        preferred_element_type=jnp.float32) * (1.0 / _HW)
    qm_ref[0] = qm


def _dn4_body(sn_ref, qt_ref, ri_ref, qm_ref, pn_ref, rcos_ref, rdn4_ref,
              out_ref):
    innerT = jax.lax.dot_general(
        sn_ref[0], qt_ref[...], (((1,), (0,)), ((), ())),
        preferred_element_type=jnp.float32)

    r = [jnp.zeros((_CHUNK, _COLS), jnp.float32) for _ in range(_K)]
    for c in range(_NCHUNK):
        v = jax.lax.slice(innerT, (c * _CHUNK, 0), ((c + 1) * _CHUNK, _COLS))
        for j in range(_K):
            nr = jnp.maximum(r[j], v)
            if j + 1 < _K:
                v = jnp.minimum(r[j], v)
            r[j] = nr
    cand = jnp.concatenate(r, axis=0)

    acc = jnp.zeros((1, _COLS), jnp.float32)
    k = jnp.full((1, _COLS), float(_K), jnp.float32)
    x = cand
    for _ in range(_K):
        m = jnp.max(x, axis=0, keepdims=True)
        eq = x == m
        cnt = jnp.sum(eq.astype(jnp.float32), axis=0, keepdims=True)
        t = jnp.minimum(cnt, k)
        acc = acc + m * t
        k = k - t
        x = jnp.where(eq, -3.4e38, x)

    accq = acc * ri_ref[0, 0:1]

    rid = jax.lax.broadcasted_iota(jnp.int32, (_QB, _COLS), 1) // _HW
    qid = jax.lax.broadcasted_iota(jnp.int32, (_QB, _COLS), 0)
    sel = (rid == qid).astype(jnp.float32)
    dn4 = jax.lax.dot_general(
        sel, jnp.transpose(accq), (((1,), (0,)), ((), ())),
        preferred_element_type=jnp.float32)

    qm = qm_ref[0]
    qmsq = jnp.sum(qm * qm, axis=1, keepdims=True)
    qmn = qm * jax.lax.rsqrt(jnp.maximum(qmsq, 1e-24))
    cos = jax.lax.dot_general(
        qmn, pn_ref[0, 0:1], (((1,), (1,)), ((), ())),
        preferred_element_type=jnp.float32)

    val = rcos_ref[0] * cos + (rdn4_ref[0] * (1.0 / _K)) * dn4
    out_ref[0] = jnp.broadcast_to(val, (_QB, 128))


def kernel(x_shot, x_query, W_enc, b_enc, r_cos, r_dn4):
    ep = x_shot.shape[0]
    xs = x_shot.reshape((-1, 3, 224, 224)).astype(jnp.bfloat16)
    xq = x_query.reshape((-1, 3, 224, 224))
    xq = jnp.pad(xq, ((0, _NQ_PAD - _NQ), (0, 0), (0, 0), (0, 0)))
    xq = xq.astype(jnp.bfloat16)
    ps = xs.reshape(25, 3, 14, 16, 14, 16).transpose(0, 2, 4, 1, 3, 5)
    ps = ps.reshape(25 * _HW, _PATCH)
    pq = xq.reshape(_NQ_PAD, 3, 14, 16, 14, 16).transpose(0, 2, 4, 1, 3, 5)
    pq = pq.reshape(_NQ_PAD * _HW, _PATCH)
    wt = W_enc.reshape(_DIM, _PATCH).T.astype(jnp.bfloat16)
    b2 = b_enc.reshape(1, _DIM)

    sn, pn = pl.pallas_call(
        _enc_support_body,
        grid=(1,),
        in_specs=[
            pl.BlockSpec((25 * _HW, _PATCH), lambda i: (0, 0)),
            pl.BlockSpec((_PATCH, _DIM), lambda i: (0, 0)),
            pl.BlockSpec((1, _DIM), lambda i: (0, 0)),
        ],
        out_specs=[
            pl.BlockSpec((_WAY, _SHW_PAD, _DIM), lambda i: (0, 0, 0)),
            pl.BlockSpec((_WAY, 8, _DIM), lambda i: (0, 0, 0)),
        ],
        out_shape=[
            jax.ShapeDtypeStruct((_WAY, _SHW_PAD, _DIM), jnp.bfloat16),
            jax.ShapeDtypeStruct((_WAY, 8, _DIM), jnp.float32),
        ],
    )(ps, wt, b2)

    nqb = _NQ_PAD // _QB
    qt, ri, qm = pl.pallas_call(
        _enc_query_body,
        grid=(nqb,),
        in_specs=[
            pl.BlockSpec((_QROWS, _PATCH), lambda i: (i, 0)),
            pl.BlockSpec((_PATCH, _DIM), lambda i: (0, 0)),
            pl.BlockSpec((1, _DIM), lambda i: (0, 0)),
        ],
        out_specs=[
            pl.BlockSpec((_DIM, _COLS), lambda i: (0, i)),
            pl.BlockSpec((1, 8, _COLS), lambda i: (i, 0, 0)),
            pl.BlockSpec((1, 8, _DIM), lambda i: (i, 0, 0)),
        ],
        out_shape=[
            jax.ShapeDtypeStruct((_DIM, nqb * _COLS), jnp.bfloat16),
            jax.ShapeDtypeStruct((nqb, 8, _COLS), jnp.float32),
            jax.ShapeDtypeStruct((nqb, 8, _DIM), jnp.float32),
        ],
    )(pq, wt, b2)

    out = pl.pallas_call(
        _dn4_body,
        grid=(_WAY, nqb),
        in_specs=[
            pl.BlockSpec((1, _SHW_PAD, _DIM), lambda w, q: (w, 0, 0)),
            pl.BlockSpec((_DIM, _COLS), lambda w, q: (0, q)),
            pl.BlockSpec((1, 8, _COLS), lambda w, q: (q, 0, 0)),
            pl.BlockSpec((1, 8, _DIM), lambda w, q: (q, 0, 0)),
            pl.BlockSpec((1, 8, _DIM), lambda w, q: (w, 0, 0)),
            pl.BlockSpec(memory_space=pltpu.SMEM),
            pl.BlockSpec(memory_space=pltpu.SMEM),
        ],
        out_specs=pl.BlockSpec((1, _QB, 128), lambda w, q: (w, q, 0)),
        out_shape=jax.ShapeDtypeStruct((_WAY, _NQ_PAD, 128), jnp.float32),
    )(sn, qt, ri, qm, pn, r_cos, r_dn4)

    logits = out[:, :_NQ, 0].T
    return logits.reshape(ep, _NQ, _WAY)


# speedup vs baseline: 42.2302x; 1.0084x over previous
import jax
import jax.numpy as jnp
from jax.experimental import pallas as pl
from jax.experimental.pallas import tpu as pltpu

_K = 5
_DIM = 384
_HW = 196
_PATCH = 768
_WAY = 5
_SHOT = 5
_NQ = 75
_NQ_PAD = 80
_QB = 16
_QROWS = _QB * _HW
_COLS = 3200
_HALF = 1664
_SHW = _SHOT * _HW
_SHW_PAD = 984
_CHUNK = 8
_NCHUNK = _SHW_PAD // _CHUNK


def _enc_support_body(p_ref, w_ref, b_ref, sn_ref, pn_ref):
    f = jax.lax.dot_general(
        p_ref[...], w_ref[...], (((1,), (0,)), ((), ())),
        preferred_element_type=jnp.float32)
    f = jnp.maximum(f + b_ref[...], 0.0)
    for w in range(_WAY):
        fw = jax.lax.slice(f, (_SHW * w, 0), (_SHW * (w + 1), _DIM))
        colsq = jnp.sum(fw * fw, axis=0, keepdims=True)
        snw = fw * jax.lax.rsqrt(colsq)
        snw = jnp.concatenate(
            [snw, jnp.zeros((_SHW_PAD - _SHW, _DIM), jnp.float32)], axis=0)
        sn_ref[w] = snw.astype(jnp.bfloat16)
        proto = jnp.sum(fw, axis=0, keepdims=True) * (1.0 / _SHW)
        pn = proto * jax.lax.rsqrt(
            jnp.maximum(jnp.sum(proto * proto), 1e-24))
        pn_ref[w] = jnp.broadcast_to(pn, (8, _DIM))


def _enc_query_body(p_ref, w_ref, b_ref, qt_ref, ri_ref, qm_ref):
    f = jax.lax.dot_general(
        p_ref[...], w_ref[...], (((1,), (0,)), ((), ())),
        preferred_element_type=jnp.float32)
    f = jnp.maximum(f + b_ref[...], 0.0)
    ft = jnp.transpose(f)
    qt_ref[...] = jnp.concatenate(
        [ft, jnp.zeros((_DIM, _COLS - _QROWS), jnp.float32)],
        axis=1).astype(jnp.bfloat16)
    rowsq = jnp.sum(ft * ft, axis=0, keepdims=True)
    ri = jnp.where(rowsq > 0.0, jax.lax.rsqrt(rowsq), 0.0)
    ri = jnp.concatenate(
        [ri, jnp.zeros((1, _COLS - _QROWS), jnp.float32)], axis=1)
    ri_ref[0] = jnp.broadcast_to(ri, (8, _COLS))
    rid = jax.lax.broadcasted_iota(jnp.int32, (_QB, _QROWS), 1) // _HW
    qid = jax.lax.broadcasted_iota(jnp.int32, (_QB, _QROWS), 0)
    sel = (rid == qid).astype(jnp.float32)
    qm = jax.lax.dot_general(
        sel, f, (((1,), (0,)), ((), ())),
        preferred_element_type=jnp.float32) * (1.0 / _HW)
    qm_ref[0] = qm


def _top5_cols(innerT, ncols):
    r = [jnp.zeros((_CHUNK, ncols), jnp.float32) for _ in range(_K)]
    for c in range(_NCHUNK):
        v = jax.lax.slice(innerT, (c * _CHUNK, 0), ((c + 1) * _CHUNK, ncols))
        for j in range(_K):
            nr = jnp.maximum(r[j], v)
            if j + 1 < _K:
                v = jnp.minimum(r[j], v)
            r[j] = nr
    cand = jnp.concatenate(r, axis=0)

    acc = jnp.zeros((1, ncols), jnp.float32)
    k = jnp.full((1, ncols), float(_K), jnp.float32)
    x = cand
    for _ in range(_K):
        m = jnp.max(x, axis=0, keepdims=True)
        eq = x == m
        cnt = jnp.sum(eq.astype(jnp.float32), axis=0, keepdims=True)
        t = jnp.minimum(cnt, k)
        acc = acc + m * t
        k = k - t
        x = jnp.where(eq, -3.4e38, x)
    return acc


def _dn4_body(sn_ref, qt_ref, ri_ref, qm_ref, pn_ref, rcos_ref, rdn4_ref,
              out_ref):
    accs = []
    for off, width in ((0, _HALF), (_HALF, _COLS - _HALF)):
        qth = jax.lax.slice(qt_ref[...], (0, off), (_DIM, off + width))
        innerT = jax.lax.dot_general(
            sn_ref[0], qth, (((1,), (0,)), ((), ())),
            preferred_element_type=jnp.float32)
        accs.append(_top5_cols(innerT, width))
    acc = jnp.concatenate(accs, axis=1)

    accq = acc * ri_ref[0, 0:1]

    rid = jax.lax.broadcasted_iota(jnp.int32, (_QB, _COLS), 1) // _HW
    qid = jax.lax.broadcasted_iota(jnp.int32, (_QB, _COLS), 0)
    sel = (rid == qid).astype(jnp.float32)
    dn4 = jax.lax.dot_general(
        sel, jnp.transpose(accq), (((1,), (0,)), ((), ())),
        preferred_element_type=jnp.float32)

    qm = qm_ref[0]
    qmsq = jnp.sum(qm * qm, axis=1, keepdims=True)
    qmn = qm * jax.lax.rsqrt(jnp.maximum(qmsq, 1e-24))
    cos = jax.lax.dot_general(
        qmn, pn_ref[0, 0:1], (((1,), (1,)), ((), ())),
        preferred_element_type=jnp.float32)

    val = rcos_ref[0] * cos + (rdn4_ref[0] * (1.0 / _K)) * dn4
    out_ref[0] = jnp.broadcast_to(val, (_QB, 128))


def kernel(x_shot, x_query, W_enc, b_enc, r_cos, r_dn4):
    ep = x_shot.shape[0]
    xs = x_shot.reshape((-1, 3, 224, 224)).astype(jnp.bfloat16)
    xq = x_query.reshape((-1, 3, 224, 224))
    xq = jnp.pad(xq, ((0, _NQ_PAD - _NQ), (0, 0), (0, 0), (0, 0)))
    xq = xq.astype(jnp.bfloat16)
    ps = xs.reshape(25, 3, 14, 16, 14, 16).transpose(0, 2, 4, 1, 3, 5)
    ps = ps.reshape(25 * _HW, _PATCH)
    pq = xq.reshape(_NQ_PAD, 3, 14, 16, 14, 16).transpose(0, 2, 4, 1, 3, 5)
    pq = pq.reshape(_NQ_PAD * _HW, _PATCH)
    wt = W_enc.reshape(_DIM, _PATCH).T.astype(jnp.bfloat16)
    b2 = b_enc.reshape(1, _DIM)

    sn, pn = pl.pallas_call(
        _enc_support_body,
        grid=(1,),
        in_specs=[
            pl.BlockSpec((25 * _HW, _PATCH), lambda i: (0, 0)),
            pl.BlockSpec((_PATCH, _DIM), lambda i: (0, 0)),
            pl.BlockSpec((1, _DIM), lambda i: (0, 0)),
        ],
        out_specs=[
            pl.BlockSpec((_WAY, _SHW_PAD, _DIM), lambda i: (0, 0, 0)),
            pl.BlockSpec((_WAY, 8, _DIM), lambda i: (0, 0, 0)),
        ],
        out_shape=[
            jax.ShapeDtypeStruct((_WAY, _SHW_PAD, _DIM), jnp.bfloat16),
            jax.ShapeDtypeStruct((_WAY, 8, _DIM), jnp.float32),
        ],
    )(ps, wt, b2)

    nqb = _NQ_PAD // _QB
    qt, ri, qm = pl.pallas_call(
        _enc_query_body,
        grid=(nqb,),
        in_specs=[
            pl.BlockSpec((_QROWS, _PATCH), lambda i: (i, 0)),
            pl.BlockSpec((_PATCH, _DIM), lambda i: (0, 0)),
            pl.BlockSpec((1, _DIM), lambda i: (0, 0)),
        ],
        out_specs=[
            pl.BlockSpec((_DIM, _COLS), lambda i: (0, i)),
            pl.BlockSpec((1, 8, _COLS), lambda i: (i, 0, 0)),
            pl.BlockSpec((1, _QB, _DIM), lambda i: (i, 0, 0)),
        ],
        out_shape=[
            jax.ShapeDtypeStruct((_DIM, nqb * _COLS), jnp.bfloat16),
            jax.ShapeDtypeStruct((nqb, 8, _COLS), jnp.float32),
            jax.ShapeDtypeStruct((nqb, _QB, _DIM), jnp.float32),
        ],
    )(pq, wt, b2)

    out = pl.pallas_call(
        _dn4_body,
        grid=(_WAY, nqb),
        in_specs=[
            pl.BlockSpec((1, _SHW_PAD, _DIM), lambda w, q: (w, 0, 0)),
            pl.BlockSpec((_DIM, _COLS), lambda w, q: (0, q)),
            pl.BlockSpec((1, 8, _COLS), lambda w, q: (q, 0, 0)),
            pl.BlockSpec((1, _QB, _DIM), lambda w, q: (q, 0, 0)),
            pl.BlockSpec((1, 8, _DIM), lambda w, q: (w, 0, 0)),
            pl.BlockSpec(memory_space=pltpu.SMEM),
            pl.BlockSpec(memory_space=pltpu.SMEM),
        ],
        out_specs=pl.BlockSpec((1, _QB, 128), lambda w, q: (w, q, 0)),
        out_shape=jax.ShapeDtypeStruct((_WAY, _NQ_PAD, 128), jnp.float32),
    )(sn, qt, ri, qm, pn, r_cos, r_dn4)

    logits = out[:, :_NQ, 0].T
    return logits.reshape(ep, _NQ, _WAY)
